# Initial kernel scaffold; baseline (speedup 1.0000x reference)
import functools

import jax
import jax.numpy as jnp
from jax import lax
from jax.experimental import pallas as pl
from jax.experimental.pallas import tpu as pltpu
from jax.experimental.pallas import tpu_sc as plsc

DIM = 64
NC, NS = 2, 16
NW = NC * NS
IDX_W = 128
CHUNK_ROWS = 512
IDX_ROWS = CHUNK_ROWS // IDX_W


@functools.lru_cache(maxsize=None)
def _build(total, vocab):
    per_w = total // NW
    n_chunks = per_w // CHUNK_ROWS
    mesh = plsc.VectorSubcoreMesh(core_axis_name="c", subcore_axis_name="s")

    @functools.partial(
        pl.kernel,
        mesh=mesh,
        out_type=jax.ShapeDtypeStruct((total, DIM), jnp.float32),
        scratch_types=[
            pltpu.VMEM((IDX_ROWS, IDX_W), jnp.int32),
            pltpu.VMEM((CHUNK_ROWS, DIM), jnp.float32),
            pltpu.SemaphoreType.DMA,
        ],
    )
    def gather_kernel(idx_hbm, table_hbm, out_hbm, idx_v, rows_v, sem):
        wid = lax.axis_index("s") * NC + lax.axis_index("c")
        row0 = wid * per_w

        def chunk(c, carry):
            off = row0 + c * CHUNK_ROWS
            pltpu.sync_copy(idx_hbm.at[pl.ds(off // IDX_W, IDX_ROWS)], idx_v)
            copies = [
                pltpu.async_copy(
                    table_hbm.at[idx_v.at[j]],
                    rows_v.at[pl.ds(j * IDX_W, IDX_W)],
                    sem,
                )
                for j in range(IDX_ROWS)
            ]
            for cp in copies:
                cp.wait()
            pltpu.sync_copy(rows_v, out_hbm.at[pl.ds(off, CHUNK_ROWS)])
            return carry

        lax.fori_loop(0, n_chunks, chunk, 0)

    return gather_kernel


def kernel(x, table):
    b, h = x.shape
    total = b * h
    idx = x.reshape(total // IDX_W, IDX_W).astype(jnp.int32)
    out = _build(total, table.shape[0])(idx, table)
    return out.reshape(b, h, DIM)


# speedup vs baseline: 1.8447x; 1.8447x over previous
import functools

import jax
import jax.numpy as jnp
from jax import lax
from jax.experimental import pallas as pl
from jax.experimental.pallas import tpu as pltpu
from jax.experimental.pallas import tpu_sc as plsc

DIM = 64
NC, NS = 2, 16
NW = NC * NS
IDX_W = 128
CHUNK_ROWS = 1024
IDX_ROWS = CHUNK_ROWS // IDX_W


@functools.lru_cache(maxsize=None)
def _build(total, vocab):
    per_w = total // NW
    n_chunks = per_w // CHUNK_ROWS
    mesh = plsc.VectorSubcoreMesh(core_axis_name="c", subcore_axis_name="s")

    @functools.partial(
        pl.kernel,
        mesh=mesh,
        out_type=jax.ShapeDtypeStruct((total, DIM), jnp.float32),
        compiler_params=pltpu.CompilerParams(use_tc_tiling_on_sc=False),
        scratch_types=[
            pltpu.VMEM((IDX_ROWS, IDX_W), jnp.int32),
            pltpu.VMEM((CHUNK_ROWS, DIM), jnp.float32),
            pltpu.SemaphoreType.DMA,
        ],
    )
    def gather_kernel(idx_hbm, table_hbm, out_hbm, idx_v, rows_v, sem):
        wid = lax.axis_index("s") * NC + lax.axis_index("c")
        row0 = wid * per_w

        def chunk(c, carry):
            off = pl.multiple_of(row0 + c * CHUNK_ROWS, CHUNK_ROWS)
            idx_row = pl.multiple_of(off // IDX_W, IDX_ROWS)
            pltpu.sync_copy(idx_hbm.at[pl.ds(idx_row, IDX_ROWS)], idx_v)
            copies = [
                pltpu.async_copy(
                    table_hbm.at[idx_v.at[j]],
                    rows_v.at[pl.ds(j * IDX_W, IDX_W)],
                    sem,
                )
                for j in range(IDX_ROWS)
            ]
            for cp in copies:
                cp.wait()
            pltpu.sync_copy(rows_v, out_hbm.at[pl.ds(off, CHUNK_ROWS)])
            return carry

        lax.fori_loop(0, n_chunks, chunk, 0)

    return gather_kernel


def kernel(x, table):
    b, h = x.shape
    total = b * h
    idx = x.reshape(total // IDX_W, IDX_W).astype(jnp.int32)
    out = _build(total, table.shape[0])(idx, table)
    return out.reshape(b, h, DIM)


# speedup vs baseline: 1.8555x; 1.0058x over previous
import functools

import jax
import jax.numpy as jnp
from jax import lax
from jax.experimental import pallas as pl
from jax.experimental.pallas import tpu as pltpu
from jax.experimental.pallas import tpu_sc as plsc

DIM = 64
NC, NS = 2, 16
NW = NC * NS
IDX_W = 128
SUB = 4
STEP_ROWS = SUB * IDX_W
NBUF = 2


@functools.lru_cache(maxsize=None)
def _build(total, vocab):
    per_w = total // NW
    n_steps = per_w // STEP_ROWS
    n_outer = n_steps // NBUF
    mesh = plsc.VectorSubcoreMesh(core_axis_name="c", subcore_axis_name="s")

    @functools.partial(
        pl.kernel,
        mesh=mesh,
        out_type=jax.ShapeDtypeStruct((total, DIM), jnp.float32),
        compiler_params=pltpu.CompilerParams(use_tc_tiling_on_sc=False),
        scratch_types=[
            pltpu.VMEM((NBUF, STEP_ROWS), jnp.int32),
            pltpu.VMEM((NBUF, STEP_ROWS, DIM), jnp.float32),
            pltpu.SemaphoreType.DMA((NBUF,)),
            pltpu.SemaphoreType.DMA((NBUF,)),
        ],
    )
    def gather_kernel(idx_hbm, table_hbm, out_hbm, idx_v, rows_v, gsem, osem):
        wid = lax.axis_index("s") * NC + lax.axis_index("c")
        row0 = wid * per_w

        def fire(step, b):
            off = pl.multiple_of(row0 + step * STEP_ROWS, STEP_ROWS)
            pltpu.sync_copy(idx_hbm.at[pl.ds(off, STEP_ROWS)], idx_v.at[b])
            for j in range(SUB):
                pltpu.async_copy(
                    table_hbm.at[idx_v.at[b, pl.ds(j * IDX_W, IDX_W)]],
                    rows_v.at[b, pl.ds(j * IDX_W, IDX_W)],
                    gsem.at[b],
                )

        def drain_gathers(b):
            for j in range(SUB):
                pltpu.make_async_copy(
                    table_hbm.at[idx_v.at[b, pl.ds(j * IDX_W, IDX_W)]],
                    rows_v.at[b, pl.ds(j * IDX_W, IDX_W)],
                    gsem.at[b],
                ).wait()

        def fire_out(step, b):
            off = pl.multiple_of(row0 + step * STEP_ROWS, STEP_ROWS)
            pltpu.async_copy(
                rows_v.at[b], out_hbm.at[pl.ds(off, STEP_ROWS)], osem.at[b]
            )

        def drain_out(step, b):
            off = pl.multiple_of(row0 + step * STEP_ROWS, STEP_ROWS)
            pltpu.make_async_copy(
                rows_v.at[b], out_hbm.at[pl.ds(off, STEP_ROWS)], osem.at[b]
            ).wait()

        fire(0, 0)

        def outer(o, carry):
            for b in range(NBUF):
                s = o * NBUF + b
                nb = (b + 1) % NBUF
                if b == NBUF - 1:
                    @pl.when(o < n_outer - 1)
                    def _():
                        drain_out(s + 1 - NBUF, nb)
                        fire(s + 1, nb)
                else:
                    @pl.when(o > 0)
                    def _():
                        drain_out(s + 1 - NBUF, nb)
                        fire(s + 1, nb)

                    @pl.when(o == 0)
                    def _():
                        fire(s + 1, nb)
                drain_gathers(b)
                fire_out(s, b)
            return carry

        lax.fori_loop(0, n_outer, outer, 0)
        for b in range(NBUF):
            drain_out(n_steps - NBUF + b, b)

    return gather_kernel


def kernel(x, table):
    b, h = x.shape
    total = b * h
    idx = x.reshape(total).astype(jnp.int32)
    out = _build(total, table.shape[0])(idx, table)
    return out.reshape(b, h, DIM)
